# baseline (speedup 1.0000x reference)
import functools

import jax
import jax.numpy as jnp
from jax import lax
from jax.experimental import pallas as pl
from jax.experimental.pallas import tpu as pltpu
from jax.experimental.pallas import tpu_sc as plsc

OUT = 7
SR = 2
B = 2
R = 128
NBOX = B * R
C = 256
NBIN = OUT * OUT
NIDX = NBIN * SR * SR * 4
NCHUNK = 7
CHUNK = NIDX // NCHUNK
SHAPES = ((128, 128), (64, 64), (32, 32), (16, 16))
SCALES = (0.25, 0.125, 0.0625, 0.03125)
BASES = (0, 32768, 40960, 43008)
NROWS = 43520
ROWS_BLK = 256
NWORKER = 32
BOX_PER_W = NBOX // NWORKER


NT = (64, 16, 4, 1)
TOFF = (0, 64, 80, 84)
NSTEP = 85


def _tab_body(x0_ref, x1_ref, x2_ref, x3_ref, out_ref):
    t = pl.program_id(1)
    br = ((t >= TOFF[1]).astype(jnp.int32) + (t >= TOFF[2]).astype(jnp.int32)
          + (t >= TOFF[3]).astype(jnp.int32))
    out_ref[...] = lax.switch(
        br,
        [lambda: x0_ref[0].T, lambda: x1_ref[0].T,
         lambda: x2_ref[0].T, lambda: x3_ref[0].T]).astype(jnp.bfloat16)


def _build_table(x0, x1, x2, x3):
    xfs = [x.reshape(B, C, -1) for x in (x0, x1, x2, x3)]
    in_specs = []
    for l in range(4):
        lo, hi = TOFF[l], TOFF[l] + NT[l] - 1
        in_specs.append(pl.BlockSpec(
            (1, C, ROWS_BLK),
            lambda b, t, _lo=lo, _hi=hi: (b, 0, jnp.clip(t, _lo, _hi) - _lo)))

    def out_map(b, t):
        rb = jnp.where(
            t < TOFF[1], b * NT[0] + t,
            jnp.where(t < TOFF[2], 128 + b * NT[1] + (t - TOFF[1]),
                      jnp.where(t < TOFF[3], 160 + b * NT[2] + (t - TOFF[2]),
                                168 + b)))
        return (rb, 0)

    return pl.pallas_call(
        _tab_body, grid=(B, NSTEP),
        in_specs=in_specs,
        out_specs=pl.BlockSpec((ROWS_BLK, C), out_map),
        out_shape=jax.ShapeDtypeStruct((NROWS, C), jnp.bfloat16))(*xfs)


PREP_BLK = 32


def _prep_body(boxes_ref, idx_ref, wt_ref):
    bf = boxes_ref[...]
    x1o = bf[:, 0:1]
    y1o = bf[:, 1:2]
    x2o = bf[:, 2:3]
    y2o = bf[:, 3:4]
    areas = (x2o - x1o) * (y2o - y1o)
    sizes = jnp.sqrt(areas)
    lvlf = jnp.floor(4.0 + jnp.log2(sizes / 224.0 + 1e-8))
    lvl = jnp.clip(lvlf, 2.0, 5.0).astype(jnp.int32) - 2

    def sel_f(v0, v1, v2, v3):
        return jnp.where(lvl == 0, v0, jnp.where(lvl == 1, v1,
                         jnp.where(lvl == 2, v2, v3)))

    scale = sel_f(*[jnp.float32(s) for s in SCALES])
    wf = sel_f(*[jnp.float32(s[1]) for s in SHAPES])
    hf = sel_f(*[jnp.float32(s[0]) for s in SHAPES])
    wi = sel_f(*[jnp.int32(s[1]) for s in SHAPES])
    basei = sel_f(*[jnp.int32(b) for b in BASES])
    hwi = sel_f(*[jnp.int32(s[0] * s[1]) for s in SHAPES])

    row0 = lax.broadcasted_iota(jnp.int32, (PREP_BLK, 1), 0)
    grow = pl.program_id(0) * PREP_BLK + row0
    bidx = lax.shift_right_logical(grow, 7)
    base = basei + bidx * hwi

    x1 = x1o * scale - 0.5
    y1 = y1o * scale - 0.5
    x2 = x2o * scale - 0.5
    y2 = y2o * scale - 0.5
    bin_w = (x2 - x1) / OUT
    bin_h = (y2 - y1) / OUT

    p = lax.broadcasted_iota(jnp.int32, (PREP_BLK, NIDX), 1)
    corner = p & 3
    bsub = lax.shift_right_logical(p, 2) & 1
    asub = lax.shift_right_logical(p, 3) & 1
    gb = lax.shift_right_logical(p, 4)
    py = gb // 7
    px = gb - py * 7

    pyf = py.astype(jnp.float32)
    pxf = px.astype(jnp.float32)
    suby = (asub.astype(jnp.float32) + 0.5) / SR
    subx = (bsub.astype(jnp.float32) + 0.5) / SR
    ys = y1 + pyf * bin_h + suby * bin_h
    xs = x1 + pxf * bin_w + subx * bin_w

    def axis(coord, limf):
        v = (coord >= -1.0) & (coord <= limf)
        c = jnp.maximum(coord, 0.0)
        c0 = jnp.floor(c)
        hi_clamp = c0 >= limf - 1.0
        lo = jnp.where(hi_clamp, limf - 1.0, c0)
        hi = jnp.where(hi_clamp, limf - 1.0, c0 + 1.0)
        ce = jnp.where(hi_clamp, limf - 1.0, c)
        lw = ce - lo
        return v, lo, hi, lw

    vy, ylo, yhi, ly = axis(ys, hf)
    vx, xlo, xhi, lx = axis(xs, wf)
    use_hiy = corner >= 2
    use_hix = (corner & 1) == 1
    wy = jnp.where(use_hiy, ly, 1.0 - ly)
    wx = jnp.where(use_hix, lx, 1.0 - lx)
    ysel = jnp.where(use_hiy, yhi, ylo).astype(jnp.int32)
    xsel = jnp.where(use_hix, xhi, xlo).astype(jnp.int32)
    idx = base + ysel * wi + xsel
    wt = jnp.where(vy & vx, wy * wx * 0.25, 0.0)
    idx_ref[...] = idx
    wt_ref[...] = wt


def _prep(boxes_flat):
    nblk = NBOX // PREP_BLK
    return pl.pallas_call(
        _prep_body, grid=(nblk,),
        in_specs=[pl.BlockSpec((PREP_BLK, 4), lambda i: (i, 0))],
        out_specs=[pl.BlockSpec((PREP_BLK, NIDX), lambda i: (i, 0)),
                   pl.BlockSpec((PREP_BLK, NIDX), lambda i: (i, 0))],
        out_shape=[jax.ShapeDtypeStruct((NBOX, NIDX), jnp.int32),
                   jax.ShapeDtypeStruct((NBOX, NIDX), jnp.float32)],
    )(boxes_flat)


def _sc_pool(table, idx3, wt):
    mesh = plsc.VectorSubcoreMesh(core_axis_name="c", subcore_axis_name="s")

    @functools.partial(
        pl.kernel, mesh=mesh,
        compiler_params=pltpu.CompilerParams(needs_layout_passes=False),
        out_type=jax.ShapeDtypeStruct((NBOX, C * NBIN), jnp.float32),
        scratch_types=[
            pltpu.VMEM((BOX_PER_W, NCHUNK, CHUNK), jnp.int32),
            pltpu.VMEM((BOX_PER_W, NIDX), jnp.float32),
            pltpu.VMEM((CHUNK, C // 2), jnp.int32),
            pltpu.VMEM((CHUNK, C // 2), jnp.int32),
            pltpu.VMEM((C * NBIN,), jnp.float32),
            pltpu.SemaphoreType.DMA,
            pltpu.SemaphoreType.DMA,
        ],
    )
    def k(table_hbm, idx_hbm, wt_hbm, out_hbm, idx_v, wt_v, rows_a, rows_b,
          acc_v, sem_a, sem_b):
        cid = lax.axis_index("c")
        sid = lax.axis_index("s")
        wid = sid * 2 + cid
        b0 = wid * BOX_PER_W
        pltpu.sync_copy(idx_hbm.at[pl.ds(b0, BOX_PER_W)], idx_v)
        pltpu.sync_copy(wt_hbm.at[pl.ds(b0, BOX_PER_W)], wt_v)

        def start_gather(ib, ch, buf, sem):
            pltpu.make_async_copy(table_hbm.at[idx_v.at[ib, ch]], buf,
                                  sem).start()

        def wait_gather(buf, sem):
            pltpu.make_async_copy(table_hbm.at[pl.ds(0, CHUNK)], buf,
                                  sem).wait()

        def compute_chunk(ib, ch, buf):
            def bin_body(bi, carry):
                gbin = ch * NCHUNK + bi
                wbase = gbin * 16
                ibv = jnp.full((16,), ib, jnp.int32)
                wspl = [
                    plsc.load_gather(
                        wt_v,
                        [ibv, jnp.full((16,), wbase + r, jnp.int32)])
                    for r in range(16)
                ]
                iot = lax.iota(jnp.int32, 16)

                def load2(row, cc):
                    ab = plsc.bitcast(buf[row, pl.ds(cc * 16, 16)],
                                      jnp.bfloat16)
                    return plsc.unpack(
                        ab, format=plsc.PackFormat.INTERLEAVED,
                        preferred_element_type=jnp.float32)

                for cc in range(8):
                    ae, ao = load2(bi * 16, cc)
                    acc_e = wspl[0] * ae
                    acc_o = wspl[0] * ao
                    for r in range(1, 16):
                        ae, ao = load2(bi * 16 + r, cc)
                        acc_e = acc_e + wspl[r] * ae
                        acc_o = acc_o + wspl[r] * ao
                    oidx = (2 * iot + cc * 32) * NBIN + gbin
                    plsc.store_scatter(acc_v, [oidx], acc_e)
                    plsc.store_scatter(acc_v, [oidx + NBIN], acc_o)
                return carry

            lax.fori_loop(0, NCHUNK, bin_body, 0)

        def box_body(ib, carry):
            def g_body(g, carry2):
                wait_gather(rows_a, sem_a)
                compute_chunk(ib, 2 * g, rows_a)
                start_gather(ib, 2 * g + 2, rows_a, sem_a)
                wait_gather(rows_b, sem_b)
                compute_chunk(ib, 2 * g + 1, rows_b)

                @pl.when(g < 2)
                def _():
                    start_gather(ib, 2 * g + 3, rows_b, sem_b)

                return carry2

            lax.fori_loop(0, 3, g_body, 0)
            wait_gather(rows_a, sem_a)
            compute_chunk(ib, 6, rows_a)

            @pl.when(ib < BOX_PER_W - 1)
            def _():
                start_gather(ib + 1, 0, rows_a, sem_a)
                start_gather(ib + 1, 1, rows_b, sem_b)

            pltpu.sync_copy(acc_v, out_hbm.at[b0 + ib])
            return carry

        start_gather(0, 0, rows_a, sem_a)
        start_gather(0, 1, rows_b, sem_b)
        lax.fori_loop(0, BOX_PER_W, box_body, 0)

    return k(table, idx3, wt)


def kernel(x0, x1, x2, x3, boxes):
    table = _build_table(x0, x1, x2, x3)
    table_i32 = lax.bitcast_convert_type(
        table.reshape(NROWS, C // 2, 2), jnp.int32)
    idx, wt = _prep(boxes.reshape(-1, 4))
    out = _sc_pool(table_i32, idx.reshape(NBOX, NCHUNK, CHUNK), wt)
    return out.reshape(NBOX, C, OUT, OUT)


# speedup vs baseline: 1.7877x; 1.7877x over previous
import functools

import jax
import jax.numpy as jnp
from jax import lax
from jax.experimental import pallas as pl
from jax.experimental.pallas import tpu as pltpu
from jax.experimental.pallas import tpu_sc as plsc

OUT = 7
SR = 2
B = 2
R = 128
NBOX = B * R
C = 256
NBIN = OUT * OUT
NIDX = NBIN * SR * SR * 4
NCHUNK = 7
CHUNK = NIDX // NCHUNK
SHAPES = ((128, 128), (64, 64), (32, 32), (16, 16))
SCALES = (0.25, 0.125, 0.0625, 0.03125)
BASES = (0, 32768, 40960, 43008)
NROWS = 43520
ROWS_BLK = 256
NWORKER = 32
BOX_PER_W = NBOX // NWORKER


NT = (64, 16, 4, 1)
TOFF = (0, 64, 80, 84)
NSTEP = 85


def _tab_body(x0_ref, x1_ref, x2_ref, x3_ref, out_ref):
    t = pl.program_id(1)
    br = ((t >= TOFF[1]).astype(jnp.int32) + (t >= TOFF[2]).astype(jnp.int32)
          + (t >= TOFF[3]).astype(jnp.int32))
    y = lax.switch(
        br,
        [lambda: x0_ref[0], lambda: x1_ref[0],
         lambda: x2_ref[0], lambda: x3_ref[0]])
    bits = lax.bitcast_convert_type(y, jnp.uint32)

    u16 = jnp.uint32(16)

    def r16(v):
        return lax.shift_right_logical(
            v + jnp.uint32(0x7FFF) + (lax.shift_right_logical(v, u16)
                                      & jnp.uint32(1)), u16)

    w = r16(bits[:C // 2, :]) | lax.shift_left(r16(bits[C // 2:, :]), u16)
    out_ref[...] = lax.bitcast_convert_type(w, jnp.int32).T


def _build_table(x0, x1, x2, x3):
    xfs = [x.reshape(B, C, -1) for x in (x0, x1, x2, x3)]
    in_specs = []
    for l in range(4):
        lo, hi = TOFF[l], TOFF[l] + NT[l] - 1
        in_specs.append(pl.BlockSpec(
            (1, C, ROWS_BLK),
            lambda b, t, _lo=lo, _hi=hi: (b, 0, jnp.clip(t, _lo, _hi) - _lo)))

    def out_map(b, t):
        rb = jnp.where(
            t < TOFF[1], b * NT[0] + t,
            jnp.where(t < TOFF[2], 128 + b * NT[1] + (t - TOFF[1]),
                      jnp.where(t < TOFF[3], 160 + b * NT[2] + (t - TOFF[2]),
                                168 + b)))
        return (rb, 0)

    return pl.pallas_call(
        _tab_body, grid=(B, NSTEP),
        in_specs=in_specs,
        out_specs=pl.BlockSpec((ROWS_BLK, C // 2), out_map),
        out_shape=jax.ShapeDtypeStruct((NROWS, C // 2), jnp.int32))(*xfs)


PREP_BLK = 32


def _prep_body(boxes_ref, idx_ref, wt_ref):
    bf = boxes_ref[...]
    x1o = bf[:, 0:1]
    y1o = bf[:, 1:2]
    x2o = bf[:, 2:3]
    y2o = bf[:, 3:4]
    areas = (x2o - x1o) * (y2o - y1o)
    sizes = jnp.sqrt(areas)
    lvlf = jnp.floor(4.0 + jnp.log2(sizes / 224.0 + 1e-8))
    lvl = jnp.clip(lvlf, 2.0, 5.0).astype(jnp.int32) - 2

    def sel_f(v0, v1, v2, v3):
        return jnp.where(lvl == 0, v0, jnp.where(lvl == 1, v1,
                         jnp.where(lvl == 2, v2, v3)))

    scale = sel_f(*[jnp.float32(s) for s in SCALES])
    wf = sel_f(*[jnp.float32(s[1]) for s in SHAPES])
    hf = sel_f(*[jnp.float32(s[0]) for s in SHAPES])
    wi = sel_f(*[jnp.int32(s[1]) for s in SHAPES])
    basei = sel_f(*[jnp.int32(b) for b in BASES])
    hwi = sel_f(*[jnp.int32(s[0] * s[1]) for s in SHAPES])

    row0 = lax.broadcasted_iota(jnp.int32, (PREP_BLK, 1), 0)
    grow = pl.program_id(0) * PREP_BLK + row0
    bidx = lax.shift_right_logical(grow, 7)
    base = basei + bidx * hwi

    x1 = x1o * scale - 0.5
    y1 = y1o * scale - 0.5
    x2 = x2o * scale - 0.5
    y2 = y2o * scale - 0.5
    bin_w = (x2 - x1) / OUT
    bin_h = (y2 - y1) / OUT

    p = lax.broadcasted_iota(jnp.int32, (PREP_BLK, NIDX), 1)
    corner = p & 3
    bsub = lax.shift_right_logical(p, 2) & 1
    asub = lax.shift_right_logical(p, 3) & 1
    gb = lax.shift_right_logical(p, 4)
    py = gb // 7
    px = gb - py * 7

    pyf = py.astype(jnp.float32)
    pxf = px.astype(jnp.float32)
    suby = (asub.astype(jnp.float32) + 0.5) / SR
    subx = (bsub.astype(jnp.float32) + 0.5) / SR
    ys = y1 + pyf * bin_h + suby * bin_h
    xs = x1 + pxf * bin_w + subx * bin_w

    def axis(coord, limf):
        v = (coord >= -1.0) & (coord <= limf)
        c = jnp.maximum(coord, 0.0)
        c0 = jnp.floor(c)
        hi_clamp = c0 >= limf - 1.0
        lo = jnp.where(hi_clamp, limf - 1.0, c0)
        hi = jnp.where(hi_clamp, limf - 1.0, c0 + 1.0)
        ce = jnp.where(hi_clamp, limf - 1.0, c)
        lw = ce - lo
        return v, lo, hi, lw

    vy, ylo, yhi, ly = axis(ys, hf)
    vx, xlo, xhi, lx = axis(xs, wf)
    use_hiy = corner >= 2
    use_hix = (corner & 1) == 1
    wy = jnp.where(use_hiy, ly, 1.0 - ly)
    wx = jnp.where(use_hix, lx, 1.0 - lx)
    ysel = jnp.where(use_hiy, yhi, ylo).astype(jnp.int32)
    xsel = jnp.where(use_hix, xhi, xlo).astype(jnp.int32)
    idx = base + ysel * wi + xsel
    wt = jnp.where(vy & vx, wy * wx * 0.25, 0.0)
    idx_ref[...] = idx
    wt_ref[...] = wt


def _prep(boxes_flat):
    nblk = NBOX // PREP_BLK
    return pl.pallas_call(
        _prep_body, grid=(nblk,),
        in_specs=[pl.BlockSpec((PREP_BLK, 4), lambda i: (i, 0))],
        out_specs=[pl.BlockSpec((PREP_BLK, NIDX), lambda i: (i, 0)),
                   pl.BlockSpec((PREP_BLK, NIDX), lambda i: (i, 0))],
        out_shape=[jax.ShapeDtypeStruct((NBOX, NIDX), jnp.int32),
                   jax.ShapeDtypeStruct((NBOX, NIDX), jnp.float32)],
    )(boxes_flat)


def _sc_pool(table, idx3, wt):
    mesh = plsc.VectorSubcoreMesh(core_axis_name="c", subcore_axis_name="s")

    @functools.partial(
        pl.kernel, mesh=mesh,
        compiler_params=pltpu.CompilerParams(needs_layout_passes=False),
        out_type=jax.ShapeDtypeStruct((NBOX, C * NBIN), jnp.float32),
        scratch_types=[
            pltpu.VMEM((BOX_PER_W, NCHUNK, CHUNK), jnp.int32),
            pltpu.VMEM((BOX_PER_W, NIDX), jnp.float32),
            pltpu.VMEM((CHUNK, C // 2), jnp.int32),
            pltpu.VMEM((CHUNK, C // 2), jnp.int32),
            pltpu.VMEM((C * NBIN,), jnp.float32),
            pltpu.SemaphoreType.DMA,
            pltpu.SemaphoreType.DMA,
        ],
    )
    def k(table_hbm, idx_hbm, wt_hbm, out_hbm, idx_v, wt_v, rows_a, rows_b,
          acc_v, sem_a, sem_b):
        cid = lax.axis_index("c")
        sid = lax.axis_index("s")
        wid = sid * 2 + cid
        b0 = wid * BOX_PER_W
        pltpu.sync_copy(idx_hbm.at[pl.ds(b0, BOX_PER_W)], idx_v)
        pltpu.sync_copy(wt_hbm.at[pl.ds(b0, BOX_PER_W)], wt_v)

        def start_gather(ib, ch, buf, sem):
            pltpu.make_async_copy(table_hbm.at[idx_v.at[ib, ch]], buf,
                                  sem).start()

        def wait_gather(buf, sem):
            pltpu.make_async_copy(table_hbm.at[pl.ds(0, CHUNK)], buf,
                                  sem).wait()

        def compute_chunk(ib, ch, buf):
            def bin_body(bi, carry):
                gbin = ch * NCHUNK + bi
                wbase = gbin * 16
                ibv = jnp.full((16,), ib, jnp.int32)
                wspl = [
                    plsc.load_gather(
                        wt_v,
                        [ibv, jnp.full((16,), wbase + r, jnp.int32)])
                    for r in range(16)
                ]
                iot = lax.iota(jnp.int32, 16)

                def load2(row, cc):
                    ab = plsc.bitcast(buf[row, pl.ds(cc * 16, 16)],
                                      jnp.bfloat16)
                    return plsc.unpack(
                        ab, format=plsc.PackFormat.INTERLEAVED,
                        preferred_element_type=jnp.float32)

                for cc in range(8):
                    ae, ao = load2(bi * 16, cc)
                    acc_e = wspl[0] * ae
                    acc_o = wspl[0] * ao
                    for r in range(1, 16):
                        ae, ao = load2(bi * 16 + r, cc)
                        acc_e = acc_e + wspl[r] * ae
                        acc_o = acc_o + wspl[r] * ao
                    oidx = (iot + cc * 16) * NBIN + gbin
                    plsc.store_scatter(acc_v, [oidx], acc_e)
                    plsc.store_scatter(acc_v, [oidx + (C // 2) * NBIN],
                                       acc_o)
                return carry

            lax.fori_loop(0, NCHUNK, bin_body, 0)

        def box_body(ib, carry):
            def g_body(g, carry2):
                wait_gather(rows_a, sem_a)
                compute_chunk(ib, 2 * g, rows_a)
                start_gather(ib, 2 * g + 2, rows_a, sem_a)
                wait_gather(rows_b, sem_b)
                compute_chunk(ib, 2 * g + 1, rows_b)

                @pl.when(g < 2)
                def _():
                    start_gather(ib, 2 * g + 3, rows_b, sem_b)

                return carry2

            lax.fori_loop(0, 3, g_body, 0)
            wait_gather(rows_a, sem_a)
            compute_chunk(ib, 6, rows_a)

            @pl.when(ib < BOX_PER_W - 1)
            def _():
                start_gather(ib + 1, 0, rows_a, sem_a)
                start_gather(ib + 1, 1, rows_b, sem_b)

            pltpu.sync_copy(acc_v, out_hbm.at[b0 + ib])
            return carry

        start_gather(0, 0, rows_a, sem_a)
        start_gather(0, 1, rows_b, sem_b)
        lax.fori_loop(0, BOX_PER_W, box_body, 0)

    return k(table, idx3, wt)


def kernel(x0, x1, x2, x3, boxes):
    table = _build_table(x0, x1, x2, x3)
    idx, wt = _prep(boxes.reshape(-1, 4))
    out = _sc_pool(table, idx.reshape(NBOX, NCHUNK, CHUNK), wt)
    return out.reshape(NBOX, C, OUT, OUT)
